# baseline (speedup 1.0000x reference)
import functools

import jax
import jax.numpy as jnp
from jax import lax
from jax.experimental import pallas as pl
from jax.experimental.pallas import tpu as pltpu
from jax.experimental.pallas import tpu_sc as plsc


def _edge_mlp_body(he_ref, w1t_ref, b1_ref, w2t_ref, b2_ref, out_ref):
    x = he_ref[...]
    y = jnp.maximum(
        jnp.dot(x, w1t_ref[...], preferred_element_type=jnp.float32) + b1_ref[...],
        0.0,
    )
    z = (
        jnp.dot(y, w2t_ref[...], preferred_element_type=jnp.float32) + b2_ref[...]
    ).astype(jnp.bfloat16)
    zi = lax.bitcast_convert_type(z, jnp.int16)
    d = zi.shape[-1]
    lo = lax.convert_element_type(zi[:, : d // 2], jnp.int32) & 0xFFFF
    hi = lax.convert_element_type(zi[:, d // 2 :], jnp.int32) << 16
    out_ref[...] = lo | hi


def _edge_mlp(he, w1t, b1, w2t, b2, block_e=2000):
    e, d = he.shape
    grid = e // block_e
    return pl.pallas_call(
        _edge_mlp_body,
        grid=(grid,),
        in_specs=[
            pl.BlockSpec((block_e, d), lambda i: (i, 0)),
            pl.BlockSpec((d, d), lambda i: (0, 0)),
            pl.BlockSpec((1, d), lambda i: (0, 0)),
            pl.BlockSpec((d, d), lambda i: (0, 0)),
            pl.BlockSpec((1, d), lambda i: (0, 0)),
        ],
        out_specs=pl.BlockSpec((block_e, d // 2), lambda i: (i, 0)),
        out_shape=jax.ShapeDtypeStruct((e, d // 2), jnp.int32),
    )(he, w1t, b1, w2t, b2)


def _pack_body(h_ref, out_ref):
    z = h_ref[...].astype(jnp.bfloat16)
    zi = lax.bitcast_convert_type(z, jnp.int16)
    d = zi.shape[-1]
    lo = lax.convert_element_type(zi[:, : d // 2], jnp.int32) & 0xFFFF
    hi = lax.convert_element_type(zi[:, d // 2 :], jnp.int32) << 16
    out_ref[...] = lo | hi


def _pack_h(h, block_n=2000):
    n, d = h.shape
    return pl.pallas_call(
        _pack_body,
        grid=(n // block_n,),
        in_specs=[pl.BlockSpec((block_n, d), lambda i: (i, 0))],
        out_specs=pl.BlockSpec((block_n, d // 2), lambda i: (i, 0)),
        out_shape=jax.ShapeDtypeStruct((n, d // 2), jnp.int32),
    )(h)


def _node_mlp_body(p0_ref, p1_ref, p2_ref, p3_ref,
                   w3t_ref, b3_ref, w4t_ref, b4_ref, out_ref):
    agg = (p0_ref[0] + p1_ref[0]) + (p2_ref[0] + p3_ref[0])
    y = jnp.maximum(
        jnp.dot(agg, w3t_ref[...], preferred_element_type=jnp.float32) + b3_ref[...],
        0.0,
    )
    out_ref[...] = (
        jnp.dot(y, w4t_ref[...], preferred_element_type=jnp.float32) + b4_ref[...]
    )


def _node_mlp(parts_a, parts_b, n, w3t, b3, w4t, b4, block_n=2000):
    d = parts_a.shape[2]
    grid = n // block_n
    return pl.pallas_call(
        _node_mlp_body,
        grid=(grid,),
        in_specs=[
            pl.BlockSpec((1, block_n, d), lambda i: (0, i, 0)),
            pl.BlockSpec((1, block_n, d), lambda i: (1, i, 0)),
            pl.BlockSpec((1, block_n, d), lambda i: (0, i, 0)),
            pl.BlockSpec((1, block_n, d), lambda i: (1, i, 0)),
            pl.BlockSpec((d, d), lambda i: (0, 0)),
            pl.BlockSpec((1, d), lambda i: (0, 0)),
            pl.BlockSpec((d, d), lambda i: (0, 0)),
            pl.BlockSpec((1, d), lambda i: (0, 0)),
        ],
        out_specs=pl.BlockSpec((block_n, d), lambda i: (i, 0)),
        out_shape=jax.ShapeDtypeStruct((n, d), jnp.float32),
    )(parts_a, parts_a, parts_b, parts_b, w3t, b3, w4t, b4)


def _sc_gather_mul_scatter(h, src3, dst3, he2, zeros_nd):
    n, d = h.shape
    nw, nchunk, ch = src3.shape
    nc, ns = 2, 16
    nbuf = 5
    ngrp = nchunk // nbuf
    n_pad = 10112
    rows_per_tile = n_pad // ns

    mesh = plsc.VectorSubcoreMesh(core_axis_name="c", subcore_axis_name="s")

    @functools.partial(
        pl.kernel,
        mesh=mesh,
        out_type=jax.ShapeDtypeStruct((nc, n_pad, d), jnp.float32),
        scratch_types=[
            pltpu.VMEM((nbuf, ch), jnp.int32),
            pltpu.VMEM((nbuf, ch), jnp.int32),
            pltpu.VMEM((nbuf, ch, d), jnp.float32),
            pltpu.VMEM((nbuf, ch, d // 2), jnp.int32),
            pltpu.VMEM_SHARED((n_pad, d), jnp.float32),
            pltpu.SemaphoreType.DMA((nbuf,)),
            pltpu.SemaphoreType.DMA((nbuf,)),
            pltpu.SemaphoreType.DMA((nbuf,)),
            pltpu.SemaphoreType.DMA((nbuf,)),
            pltpu.SemaphoreType.DMA((nbuf,)),
        ],
    )
    def k(h_hbm, src_hbm, dst_hbm, he2_hbm, z_hbm, out_hbm,
          sidx, didx, hrows, herows, agg,
          sem_si, sem_di, sem_g, sem_e, sem_s):
        cid = lax.axis_index("c")
        sid = lax.axis_index("s")
        wid = sid * nc + cid
        ebase = wid * nchunk * ch
        row0 = sid * rows_per_tile

        def issue_idx(c, b):
            pltpu.async_copy(src_hbm.at[wid, c], sidx.at[b], sem_si.at[b])
            pltpu.async_copy(dst_hbm.at[wid, c], didx.at[b], sem_di.at[b])

        def wait_idx(c, b):
            pltpu.make_async_copy(
                src_hbm.at[wid, c], sidx.at[b], sem_si.at[b]
            ).wait()
            pltpu.make_async_copy(
                dst_hbm.at[wid, c], didx.at[b], sem_di.at[b]
            ).wait()

        def issue_in(c, b):
            pltpu.async_copy(h_hbm.at[sidx.at[b]], hrows.at[b], sem_g.at[b])
            pltpu.async_copy(
                he2_hbm.at[pl.ds(ebase + c * ch, ch)], herows.at[b], sem_e.at[b]
            )

        def wait_in(c, b):
            pltpu.make_async_copy(
                h_hbm.at[sidx.at[b]], hrows.at[b], sem_g.at[b]
            ).wait()
            pltpu.make_async_copy(
                he2_hbm.at[pl.ds(ebase + c * ch, ch)], herows.at[b], sem_e.at[b]
            ).wait()

        def issue_scat(c, b):
            pltpu.async_copy(
                hrows.at[b], agg.at[didx.at[b]], sem_s.at[b], add=True
            )

        def wait_scat(c, b):
            pltpu.make_async_copy(
                hrows.at[b], agg.at[didx.at[b]], sem_s.at[b]
            ).wait()

        def mul_buf(b):
            def mul_row(i, carry):
                for t in range(d // 32):
                    v = herows[b, i, pl.ds(t * 16, 16)]
                    lo = lax.bitcast_convert_type(lax.shift_left(v, 16), jnp.float32)
                    hi = lax.bitcast_convert_type(
                        lax.bitwise_and(v, jnp.int32(-65536)), jnp.float32
                    )
                    sl0 = pl.ds(t * 16, 16)
                    sl1 = pl.ds(d // 2 + t * 16, 16)
                    hrows[b, i, sl0] = hrows[b, i, sl0] * lo
                    hrows[b, i, sl1] = hrows[b, i, sl1] * hi
                return carry

            lax.fori_loop(0, ch, mul_row, 0, unroll=4)

        issue_idx(0, 0)
        issue_idx(1, 1)
        pltpu.sync_copy(
            z_hbm.at[pl.ds(0, rows_per_tile)],
            agg.at[pl.ds(row0, rows_per_tile)],
        )
        wait_idx(0, 0)
        issue_in(0, 0)
        plsc.subcore_barrier()

        def group(g, carry):
            for u in range(nbuf):
                c = g * nbuf + u

                @pl.when(c >= 3)
                def _():
                    wait_scat(c - 3, (u - 3) % nbuf)

                @pl.when(c + 2 < nchunk)
                def _():
                    issue_idx(c + 2, (u + 2) % nbuf)

                @pl.when(c + 1 < nchunk)
                def _():
                    wait_idx(c + 1, (u + 1) % nbuf)
                    issue_in(c + 1, (u + 1) % nbuf)

                wait_in(c, u)
                mul_buf(u)
                issue_scat(c, u)
            return carry

        lax.fori_loop(0, ngrp, group, 0)
        for t3 in range(3):
            c = nchunk - 3 + t3
            wait_scat(c, c % nbuf)
        plsc.subcore_barrier()

        pltpu.sync_copy(
            agg.at[pl.ds(row0, rows_per_tile)],
            out_hbm.at[cid, pl.ds(row0, rows_per_tile)],
        )

    return k(h, src3, dst3, he2, zeros_nd)


def kernel(h, edge_index, he, W1, b1, W2, b2, W3, b3, W4, b4):
    n, d = h.shape
    e = he.shape[0]
    nw, ch = 32, 16
    ea = 163840
    w1t, b1r = W1.T, b1.reshape(1, d)
    w2t, b2r = W2.T, b2.reshape(1, d)
    zeros_nd = jnp.zeros((n, d), jnp.float32)
    parts = []
    for lo_e, hi_e in ((0, ea), (ea, e)):
        ee = hi_e - lo_e
        nchunk = ee // (nw * ch)
        src3 = edge_index[0, lo_e:hi_e].reshape(nw, nchunk, ch)
        dst3 = edge_index[1, lo_e:hi_e].reshape(nw, nchunk, ch)
        he2 = _edge_mlp(he[lo_e:hi_e], w1t, b1r, w2t, b2r, block_e=2560)
        parts.append(_sc_gather_mul_scatter(h, src3, dst3, he2, zeros_nd))
    out = _node_mlp(parts[0], parts[1], n,
                    W3.T, b3.reshape(1, d), W4.T, b4.reshape(1, d))
    return out


# speedup vs baseline: 1.0174x; 1.0174x over previous
import functools

import jax
import jax.numpy as jnp
from jax import lax
from jax.experimental import pallas as pl
from jax.experimental.pallas import tpu as pltpu
from jax.experimental.pallas import tpu_sc as plsc


def _edge_mlp_body(he_ref, w1t_ref, b1_ref, w2t_ref, b2_ref, out_ref):
    x = he_ref[...]
    y = jnp.maximum(
        jnp.dot(x, w1t_ref[...], preferred_element_type=jnp.float32) + b1_ref[...],
        0.0,
    )
    z = (
        jnp.dot(y, w2t_ref[...], preferred_element_type=jnp.float32) + b2_ref[...]
    ).astype(jnp.bfloat16)
    zi = lax.bitcast_convert_type(z, jnp.int16)
    d = zi.shape[-1]
    lo = lax.convert_element_type(zi[:, : d // 2], jnp.int32) & 0xFFFF
    hi = lax.convert_element_type(zi[:, d // 2 :], jnp.int32) << 16
    out_ref[...] = lo | hi


def _edge_mlp(he, w1t, b1, w2t, b2, block_e=2000):
    e, d = he.shape
    grid = e // block_e
    return pl.pallas_call(
        _edge_mlp_body,
        grid=(grid,),
        in_specs=[
            pl.BlockSpec((block_e, d), lambda i: (i, 0)),
            pl.BlockSpec((d, d), lambda i: (0, 0)),
            pl.BlockSpec((1, d), lambda i: (0, 0)),
            pl.BlockSpec((d, d), lambda i: (0, 0)),
            pl.BlockSpec((1, d), lambda i: (0, 0)),
        ],
        out_specs=pl.BlockSpec((block_e, d // 2), lambda i: (i, 0)),
        out_shape=jax.ShapeDtypeStruct((e, d // 2), jnp.int32),
    )(he, w1t, b1, w2t, b2)


def _node_mlp_body(p0_ref, p1_ref, w3t_ref, b3_ref, w4t_ref, b4_ref, out_ref):
    agg = p0_ref[0] + p1_ref[0]
    y = jnp.maximum(
        jnp.dot(agg, w3t_ref[...], preferred_element_type=jnp.float32) + b3_ref[...],
        0.0,
    )
    out_ref[...] = (
        jnp.dot(y, w4t_ref[...], preferred_element_type=jnp.float32) + b4_ref[...]
    )


def _node_mlp(parts, n, w3t, b3, w4t, b4, block_n=2000):
    d = parts.shape[2]
    grid = n // block_n
    return pl.pallas_call(
        _node_mlp_body,
        grid=(grid,),
        in_specs=[
            pl.BlockSpec((1, block_n, d), lambda i: (0, i, 0)),
            pl.BlockSpec((1, block_n, d), lambda i: (1, i, 0)),
            pl.BlockSpec((d, d), lambda i: (0, 0)),
            pl.BlockSpec((1, d), lambda i: (0, 0)),
            pl.BlockSpec((d, d), lambda i: (0, 0)),
            pl.BlockSpec((1, d), lambda i: (0, 0)),
        ],
        out_specs=pl.BlockSpec((block_n, d), lambda i: (i, 0)),
        out_shape=jax.ShapeDtypeStruct((n, d), jnp.float32),
    )(parts, parts, w3t, b3, w4t, b4)


def _sc_gather_mul_scatter(h, src3, dst3, he2, zeros_nd):
    n, d = h.shape
    nw, nchunk, ch = src3.shape
    nc, ns = 2, 16
    nbuf = 5
    ngrp = nchunk // nbuf
    n_pad = 10112
    rows_per_tile = n_pad // ns

    mesh = plsc.VectorSubcoreMesh(core_axis_name="c", subcore_axis_name="s")

    @functools.partial(
        pl.kernel,
        mesh=mesh,
        out_type=jax.ShapeDtypeStruct((nc, n_pad, d), jnp.float32),
        scratch_types=[
            pltpu.VMEM((nbuf, ch), jnp.int32),
            pltpu.VMEM((nbuf, ch), jnp.int32),
            pltpu.VMEM((nbuf, ch, d), jnp.float32),
            pltpu.VMEM((nbuf, ch, d // 2), jnp.int32),
            pltpu.VMEM_SHARED((n_pad, d), jnp.float32),
            pltpu.SemaphoreType.DMA((nbuf,)),
            pltpu.SemaphoreType.DMA((nbuf,)),
            pltpu.SemaphoreType.DMA((nbuf,)),
            pltpu.SemaphoreType.DMA((nbuf,)),
            pltpu.SemaphoreType.DMA((nbuf,)),
        ],
    )
    def k(h_hbm, src_hbm, dst_hbm, he2_hbm, z_hbm, out_hbm,
          sidx, didx, hrows, herows, agg,
          sem_si, sem_di, sem_g, sem_e, sem_s):
        cid = lax.axis_index("c")
        sid = lax.axis_index("s")
        wid = sid * nc + cid
        ebase = wid * nchunk * ch
        row0 = sid * rows_per_tile

        def issue_idx(c, b):
            pltpu.async_copy(src_hbm.at[wid, c], sidx.at[b], sem_si.at[b])
            pltpu.async_copy(dst_hbm.at[wid, c], didx.at[b], sem_di.at[b])

        def wait_idx(c, b):
            pltpu.make_async_copy(
                src_hbm.at[wid, c], sidx.at[b], sem_si.at[b]
            ).wait()
            pltpu.make_async_copy(
                dst_hbm.at[wid, c], didx.at[b], sem_di.at[b]
            ).wait()

        def issue_in(c, b):
            pltpu.async_copy(h_hbm.at[sidx.at[b]], hrows.at[b], sem_g.at[b])
            pltpu.async_copy(
                he2_hbm.at[pl.ds(ebase + c * ch, ch)], herows.at[b], sem_e.at[b]
            )

        def wait_in(c, b):
            pltpu.make_async_copy(
                h_hbm.at[sidx.at[b]], hrows.at[b], sem_g.at[b]
            ).wait()
            pltpu.make_async_copy(
                he2_hbm.at[pl.ds(ebase + c * ch, ch)], herows.at[b], sem_e.at[b]
            ).wait()

        def issue_scat(c, b):
            pltpu.async_copy(
                hrows.at[b], agg.at[didx.at[b]], sem_s.at[b], add=True
            )

        def wait_scat(c, b):
            pltpu.make_async_copy(
                hrows.at[b], agg.at[didx.at[b]], sem_s.at[b]
            ).wait()

        def mul_buf(b):
            def mul_row(i, carry):
                for t in range(d // 32):
                    v = herows[b, i, pl.ds(t * 16, 16)]
                    lo = lax.bitcast_convert_type(lax.shift_left(v, 16), jnp.float32)
                    hi = lax.bitcast_convert_type(
                        lax.bitwise_and(v, jnp.int32(-65536)), jnp.float32
                    )
                    sl0 = pl.ds(t * 16, 16)
                    sl1 = pl.ds(d // 2 + t * 16, 16)
                    hrows[b, i, sl0] = hrows[b, i, sl0] * lo
                    hrows[b, i, sl1] = hrows[b, i, sl1] * hi
                return carry

            lax.fori_loop(0, ch, mul_row, 0, unroll=4)

        issue_idx(0, 0)
        issue_idx(1, 1)
        pltpu.sync_copy(
            z_hbm.at[pl.ds(0, rows_per_tile)],
            agg.at[pl.ds(row0, rows_per_tile)],
        )
        wait_idx(0, 0)
        issue_in(0, 0)
        plsc.subcore_barrier()

        def group(g, carry):
            for u in range(nbuf):
                c = g * nbuf + u

                @pl.when(c >= 3)
                def _():
                    wait_scat(c - 3, (u - 3) % nbuf)

                @pl.when(c + 2 < nchunk)
                def _():
                    issue_idx(c + 2, (u + 2) % nbuf)

                @pl.when(c + 1 < nchunk)
                def _():
                    wait_idx(c + 1, (u + 1) % nbuf)
                    issue_in(c + 1, (u + 1) % nbuf)

                wait_in(c, u)
                mul_buf(u)
                issue_scat(c, u)
            return carry

        lax.fori_loop(0, ngrp, group, 0)
        for t3 in range(3):
            c = nchunk - 3 + t3
            wait_scat(c, c % nbuf)
        plsc.subcore_barrier()

        pltpu.sync_copy(
            agg.at[pl.ds(row0, rows_per_tile)],
            out_hbm.at[cid, pl.ds(row0, rows_per_tile)],
        )

    return k(h, src3, dst3, he2, zeros_nd)


def kernel(h, edge_index, he, W1, b1, W2, b2, W3, b3, W4, b4):
    n, d = h.shape
    e = he.shape[0]
    nw, ch = 32, 16
    nchunk = e // (nw * ch)
    src3 = edge_index[0].reshape(nw, nchunk, ch)
    dst3 = edge_index[1].reshape(nw, nchunk, ch)
    he2 = _edge_mlp(he, W1.T, b1.reshape(1, d), W2.T, b2.reshape(1, d))
    zeros_nd = jnp.zeros((n, d), jnp.float32)
    parts = _sc_gather_mul_scatter(h, src3, dst3, he2, zeros_nd)
    out = _node_mlp(parts, n, W3.T, b3.reshape(1, d), W4.T, b4.reshape(1, d))
    return out


# speedup vs baseline: 1.1101x; 1.0911x over previous
import functools

import jax
import jax.numpy as jnp
from jax import lax
from jax.experimental import pallas as pl
from jax.experimental.pallas import tpu as pltpu
from jax.experimental.pallas import tpu_sc as plsc


def _edge_mlp_body(he_ref, w1t_ref, b1_ref, w2t_ref, b2_ref, out_ref):
    x = he_ref[...]
    y = jnp.maximum(
        jnp.dot(x, w1t_ref[...], preferred_element_type=jnp.float32) + b1_ref[...],
        0.0,
    )
    z = (
        jnp.dot(y, w2t_ref[...], preferred_element_type=jnp.float32) + b2_ref[...]
    ).astype(jnp.bfloat16)
    zi = lax.bitcast_convert_type(z, jnp.int16)
    d = zi.shape[-1]
    lo = lax.convert_element_type(zi[:, : d // 2], jnp.int32) & 0xFFFF
    hi = lax.convert_element_type(zi[:, d // 2 :], jnp.int32) << 16
    out_ref[...] = lo | hi


def _edge_mlp(he, w1t, b1, w2t, b2, block_e=4000):
    e, d = he.shape
    grid = e // block_e
    return pl.pallas_call(
        _edge_mlp_body,
        grid=(grid,),
        in_specs=[
            pl.BlockSpec((block_e, d), lambda i: (i, 0)),
            pl.BlockSpec((d, d), lambda i: (0, 0)),
            pl.BlockSpec((1, d), lambda i: (0, 0)),
            pl.BlockSpec((d, d), lambda i: (0, 0)),
            pl.BlockSpec((1, d), lambda i: (0, 0)),
        ],
        out_specs=pl.BlockSpec((block_e, d // 2), lambda i: (i, 0)),
        out_shape=jax.ShapeDtypeStruct((e, d // 2), jnp.int32),
    )(he, w1t, b1, w2t, b2)


def _node_mlp_body(p0_ref, p1_ref, w3t_ref, b3_ref, w4t_ref, b4_ref, out_ref):
    agg = p0_ref[0] + p1_ref[0]
    y = jnp.maximum(
        jnp.dot(agg, w3t_ref[...], preferred_element_type=jnp.float32) + b3_ref[...],
        0.0,
    )
    out_ref[...] = (
        jnp.dot(y, w4t_ref[...], preferred_element_type=jnp.float32) + b4_ref[...]
    )


def _node_mlp(parts, n, w3t, b3, w4t, b4, block_n=2000):
    d = parts.shape[2]
    grid = n // block_n
    return pl.pallas_call(
        _node_mlp_body,
        grid=(grid,),
        in_specs=[
            pl.BlockSpec((1, block_n, d), lambda i: (0, i, 0)),
            pl.BlockSpec((1, block_n, d), lambda i: (1, i, 0)),
            pl.BlockSpec((d, d), lambda i: (0, 0)),
            pl.BlockSpec((1, d), lambda i: (0, 0)),
            pl.BlockSpec((d, d), lambda i: (0, 0)),
            pl.BlockSpec((1, d), lambda i: (0, 0)),
        ],
        out_specs=pl.BlockSpec((block_n, d), lambda i: (i, 0)),
        out_shape=jax.ShapeDtypeStruct((n, d), jnp.float32),
    )(parts, parts, w3t, b3, w4t, b4)


def _sc_gather_mul_scatter(h, src3, dst3, he2, zeros_nd):
    n, d = h.shape
    nw, nchunk, ch = src3.shape
    nc, ns = 2, 16
    nbuf = 5
    ngrp = nchunk // nbuf
    n_pad = 10112
    rows_per_tile = n_pad // ns

    mesh = plsc.VectorSubcoreMesh(core_axis_name="c", subcore_axis_name="s")

    @functools.partial(
        pl.kernel,
        mesh=mesh,
        out_type=jax.ShapeDtypeStruct((nc, n_pad, d), jnp.float32),
        scratch_types=[
            pltpu.VMEM((nbuf, ch), jnp.int32),
            pltpu.VMEM((nbuf, ch), jnp.int32),
            pltpu.VMEM((nbuf, ch, d), jnp.float32),
            pltpu.VMEM((nbuf, ch, d // 2), jnp.int32),
            pltpu.VMEM_SHARED((n_pad, d), jnp.float32),
            pltpu.SemaphoreType.DMA((nbuf,)),
            pltpu.SemaphoreType.DMA((nbuf,)),
            pltpu.SemaphoreType.DMA((nbuf,)),
            pltpu.SemaphoreType.DMA((nbuf,)),
            pltpu.SemaphoreType.DMA((nbuf,)),
        ],
    )
    def k(h_hbm, src_hbm, dst_hbm, he2_hbm, z_hbm, out_hbm,
          sidx, didx, hrows, herows, agg,
          sem_si, sem_di, sem_g, sem_e, sem_s):
        cid = lax.axis_index("c")
        sid = lax.axis_index("s")
        wid = sid * nc + cid
        ebase = wid * nchunk * ch
        row0 = sid * rows_per_tile

        def issue_idx(c, b):
            pltpu.async_copy(src_hbm.at[wid, c], sidx.at[b], sem_si.at[b])
            pltpu.async_copy(dst_hbm.at[wid, c], didx.at[b], sem_di.at[b])

        def wait_idx(c, b):
            pltpu.make_async_copy(
                src_hbm.at[wid, c], sidx.at[b], sem_si.at[b]
            ).wait()
            pltpu.make_async_copy(
                dst_hbm.at[wid, c], didx.at[b], sem_di.at[b]
            ).wait()

        def issue_in(c, b):
            pltpu.async_copy(h_hbm.at[sidx.at[b]], hrows.at[b], sem_g.at[b])
            pltpu.async_copy(
                he2_hbm.at[pl.ds(ebase + c * ch, ch)], herows.at[b], sem_e.at[b]
            )

        def wait_in(c, b):
            pltpu.make_async_copy(
                h_hbm.at[sidx.at[b]], hrows.at[b], sem_g.at[b]
            ).wait()
            pltpu.make_async_copy(
                he2_hbm.at[pl.ds(ebase + c * ch, ch)], herows.at[b], sem_e.at[b]
            ).wait()

        def issue_scat(c, b):
            pltpu.async_copy(
                hrows.at[b], agg.at[didx.at[b]], sem_s.at[b], add=True
            )

        def wait_scat(c, b):
            pltpu.make_async_copy(
                hrows.at[b], agg.at[didx.at[b]], sem_s.at[b]
            ).wait()

        def mul_buf(b):
            def mul_row(i, carry):
                for t in range(d // 32):
                    v = herows[b, i, pl.ds(t * 16, 16)]
                    lo = lax.bitcast_convert_type(lax.shift_left(v, 16), jnp.float32)
                    hi = lax.bitcast_convert_type(
                        lax.bitwise_and(v, jnp.int32(-65536)), jnp.float32
                    )
                    sl0 = pl.ds(t * 16, 16)
                    sl1 = pl.ds(d // 2 + t * 16, 16)
                    hrows[b, i, sl0] = hrows[b, i, sl0] * lo
                    hrows[b, i, sl1] = hrows[b, i, sl1] * hi
                return carry

            lax.fori_loop(0, ch, mul_row, 0, unroll=4)

        issue_idx(0, 0)
        issue_idx(1, 1)
        pltpu.sync_copy(
            z_hbm.at[pl.ds(0, rows_per_tile)],
            agg.at[pl.ds(row0, rows_per_tile)],
        )
        wait_idx(0, 0)
        issue_in(0, 0)
        plsc.subcore_barrier()

        def group(g, carry):
            for u in range(nbuf):
                c = g * nbuf + u

                @pl.when(c >= 3)
                def _():
                    wait_scat(c - 3, (u - 3) % nbuf)

                @pl.when(c + 2 < nchunk)
                def _():
                    issue_idx(c + 2, (u + 2) % nbuf)

                @pl.when(c + 1 < nchunk)
                def _():
                    wait_idx(c + 1, (u + 1) % nbuf)
                    issue_in(c + 1, (u + 1) % nbuf)

                wait_in(c, u)
                mul_buf(u)
                issue_scat(c, u)
            return carry

        lax.fori_loop(0, ngrp, group, 0)
        for t3 in range(3):
            c = nchunk - 3 + t3
            wait_scat(c, c % nbuf)
        plsc.subcore_barrier()

        pltpu.sync_copy(
            agg.at[pl.ds(row0, rows_per_tile)],
            out_hbm.at[cid, pl.ds(row0, rows_per_tile)],
        )

    return k(h, src3, dst3, he2, zeros_nd)


def kernel(h, edge_index, he, W1, b1, W2, b2, W3, b3, W4, b4):
    n, d = h.shape
    e = he.shape[0]
    nw, ch = 32, 16
    nchunk = e // (nw * ch)
    src3 = edge_index[0].reshape(nw, nchunk, ch)
    dst3 = edge_index[1].reshape(nw, nchunk, ch)
    he2 = _edge_mlp(he, W1.T, b1.reshape(1, d), W2.T, b2.reshape(1, d))
    zeros_nd = jnp.zeros((n, d), jnp.float32)
    parts = _sc_gather_mul_scatter(h, src3, dst3, he2, zeros_nd)
    out = _node_mlp(parts, n, W3.T, b3.reshape(1, d), W4.T, b4.reshape(1, d))
    return out


# speedup vs baseline: 1.1641x; 1.0487x over previous
import functools

import jax
import jax.numpy as jnp
from jax import lax
from jax.experimental import pallas as pl
from jax.experimental.pallas import tpu as pltpu
from jax.experimental.pallas import tpu_sc as plsc


def _edge_mlp_body(he_ref, w1t_ref, b1_ref, w2t_ref, b2_ref, out_ref):
    x = he_ref[...]
    y = jnp.maximum(
        jnp.dot(x, w1t_ref[...], preferred_element_type=jnp.float32) + b1_ref[...],
        0.0,
    )
    z = (
        jnp.dot(y, w2t_ref[...], preferred_element_type=jnp.float32) + b2_ref[...]
    ).astype(jnp.bfloat16)
    zi = lax.bitcast_convert_type(z, jnp.int16)
    d = zi.shape[-1]
    lo = lax.convert_element_type(zi[:, : d // 2], jnp.int32) & 0xFFFF
    hi = lax.convert_element_type(zi[:, d // 2 :], jnp.int32) << 16
    out_ref[...] = lo | hi


def _edge_mlp(he, w1t, b1, w2t, b2, block_e=8000):
    e, d = he.shape
    grid = e // block_e
    return pl.pallas_call(
        _edge_mlp_body,
        grid=(grid,),
        in_specs=[
            pl.BlockSpec((block_e, d), lambda i: (i, 0)),
            pl.BlockSpec((d, d), lambda i: (0, 0)),
            pl.BlockSpec((1, d), lambda i: (0, 0)),
            pl.BlockSpec((d, d), lambda i: (0, 0)),
            pl.BlockSpec((1, d), lambda i: (0, 0)),
        ],
        out_specs=pl.BlockSpec((block_e, d // 2), lambda i: (i, 0)),
        out_shape=jax.ShapeDtypeStruct((e, d // 2), jnp.int32),
    )(he, w1t, b1, w2t, b2)


def _node_mlp_body(p0_ref, p1_ref, w3t_ref, b3_ref, w4t_ref, b4_ref, out_ref):
    agg = p0_ref[0] + p1_ref[0]
    y = jnp.maximum(
        jnp.dot(agg, w3t_ref[...], preferred_element_type=jnp.float32) + b3_ref[...],
        0.0,
    )
    out_ref[...] = (
        jnp.dot(y, w4t_ref[...], preferred_element_type=jnp.float32) + b4_ref[...]
    )


def _node_mlp(parts, n, w3t, b3, w4t, b4, block_n=2000):
    d = parts.shape[2]
    grid = n // block_n
    return pl.pallas_call(
        _node_mlp_body,
        grid=(grid,),
        in_specs=[
            pl.BlockSpec((1, block_n, d), lambda i: (0, i, 0)),
            pl.BlockSpec((1, block_n, d), lambda i: (1, i, 0)),
            pl.BlockSpec((d, d), lambda i: (0, 0)),
            pl.BlockSpec((1, d), lambda i: (0, 0)),
            pl.BlockSpec((d, d), lambda i: (0, 0)),
            pl.BlockSpec((1, d), lambda i: (0, 0)),
        ],
        out_specs=pl.BlockSpec((block_n, d), lambda i: (i, 0)),
        out_shape=jax.ShapeDtypeStruct((n, d), jnp.float32),
    )(parts, parts, w3t, b3, w4t, b4)


def _sc_gather_mul_scatter(h, src3, dst3, he2, zeros_nd):
    n, d = h.shape
    nw, nchunk, ch = src3.shape
    nc, ns = 2, 16
    nbuf = 5
    ngrp = nchunk // nbuf
    n_pad = 10112
    rows_per_tile = n_pad // ns

    mesh = plsc.VectorSubcoreMesh(core_axis_name="c", subcore_axis_name="s")

    @functools.partial(
        pl.kernel,
        mesh=mesh,
        out_type=jax.ShapeDtypeStruct((nc, n_pad, d), jnp.float32),
        scratch_types=[
            pltpu.VMEM((nbuf, ch), jnp.int32),
            pltpu.VMEM((nbuf, ch), jnp.int32),
            pltpu.VMEM((nbuf, ch, d), jnp.float32),
            pltpu.VMEM((nbuf, ch, d // 2), jnp.int32),
            pltpu.VMEM_SHARED((n_pad, d), jnp.float32),
            pltpu.SemaphoreType.DMA((nbuf,)),
            pltpu.SemaphoreType.DMA((nbuf,)),
            pltpu.SemaphoreType.DMA((nbuf,)),
            pltpu.SemaphoreType.DMA((nbuf,)),
            pltpu.SemaphoreType.DMA((nbuf,)),
        ],
    )
    def k(h_hbm, src_hbm, dst_hbm, he2_hbm, z_hbm, out_hbm,
          sidx, didx, hrows, herows, agg,
          sem_si, sem_di, sem_g, sem_e, sem_s):
        cid = lax.axis_index("c")
        sid = lax.axis_index("s")
        wid = sid * nc + cid
        ebase = wid * nchunk * ch
        row0 = sid * rows_per_tile

        def issue_idx(c, b):
            pltpu.async_copy(src_hbm.at[wid, c], sidx.at[b], sem_si.at[b])
            pltpu.async_copy(dst_hbm.at[wid, c], didx.at[b], sem_di.at[b])

        def wait_idx(c, b):
            pltpu.make_async_copy(
                src_hbm.at[wid, c], sidx.at[b], sem_si.at[b]
            ).wait()
            pltpu.make_async_copy(
                dst_hbm.at[wid, c], didx.at[b], sem_di.at[b]
            ).wait()

        def issue_in(c, b):
            pltpu.async_copy(h_hbm.at[sidx.at[b]], hrows.at[b], sem_g.at[b])
            pltpu.async_copy(
                he2_hbm.at[pl.ds(ebase + c * ch, ch)], herows.at[b], sem_e.at[b]
            )

        def wait_in(c, b):
            pltpu.make_async_copy(
                h_hbm.at[sidx.at[b]], hrows.at[b], sem_g.at[b]
            ).wait()
            pltpu.make_async_copy(
                he2_hbm.at[pl.ds(ebase + c * ch, ch)], herows.at[b], sem_e.at[b]
            ).wait()

        def issue_scat(c, b):
            pltpu.async_copy(
                hrows.at[b], agg.at[didx.at[b]], sem_s.at[b], add=True
            )

        def wait_scat(c, b):
            pltpu.make_async_copy(
                hrows.at[b], agg.at[didx.at[b]], sem_s.at[b]
            ).wait()

        def mul_buf(b):
            def mul_row(i, carry):
                for t in range(d // 32):
                    v = herows[b, i, pl.ds(t * 16, 16)]
                    lo = lax.bitcast_convert_type(lax.shift_left(v, 16), jnp.float32)
                    hi = lax.bitcast_convert_type(
                        lax.bitwise_and(v, jnp.int32(-65536)), jnp.float32
                    )
                    sl0 = pl.ds(t * 16, 16)
                    sl1 = pl.ds(d // 2 + t * 16, 16)
                    hrows[b, i, sl0] = hrows[b, i, sl0] * lo
                    hrows[b, i, sl1] = hrows[b, i, sl1] * hi
                return carry

            lax.fori_loop(0, ch, mul_row, 0, unroll=4)

        issue_idx(0, 0)
        issue_idx(1, 1)
        pltpu.sync_copy(
            z_hbm.at[pl.ds(0, rows_per_tile)],
            agg.at[pl.ds(row0, rows_per_tile)],
        )
        wait_idx(0, 0)
        issue_in(0, 0)
        plsc.subcore_barrier()

        def group(g, carry):
            for u in range(nbuf):
                c = g * nbuf + u

                @pl.when(c >= 3)
                def _():
                    wait_scat(c - 3, (u - 3) % nbuf)

                @pl.when(c + 2 < nchunk)
                def _():
                    issue_idx(c + 2, (u + 2) % nbuf)

                @pl.when(c + 1 < nchunk)
                def _():
                    wait_idx(c + 1, (u + 1) % nbuf)
                    issue_in(c + 1, (u + 1) % nbuf)

                wait_in(c, u)
                mul_buf(u)
                issue_scat(c, u)
            return carry

        lax.fori_loop(0, ngrp, group, 0)
        for t3 in range(3):
            c = nchunk - 3 + t3
            wait_scat(c, c % nbuf)
        plsc.subcore_barrier()

        pltpu.sync_copy(
            agg.at[pl.ds(row0, rows_per_tile)],
            out_hbm.at[cid, pl.ds(row0, rows_per_tile)],
        )

    return k(h, src3, dst3, he2, zeros_nd)


def kernel(h, edge_index, he, W1, b1, W2, b2, W3, b3, W4, b4):
    n, d = h.shape
    e = he.shape[0]
    nw, ch = 32, 16
    nchunk = e // (nw * ch)
    src3 = edge_index[0].reshape(nw, nchunk, ch)
    dst3 = edge_index[1].reshape(nw, nchunk, ch)
    he2 = _edge_mlp(he, W1.T, b1.reshape(1, d), W2.T, b2.reshape(1, d))
    zeros_nd = jnp.zeros((n, d), jnp.float32)
    parts = _sc_gather_mul_scatter(h, src3, dst3, he2, zeros_nd)
    out = _node_mlp(parts, n, W3.T, b3.reshape(1, d), W4.T, b4.reshape(1, d))
    return out


# speedup vs baseline: 1.1839x; 1.0170x over previous
import functools

import jax
import jax.numpy as jnp
from jax import lax
from jax.experimental import pallas as pl
from jax.experimental.pallas import tpu as pltpu
from jax.experimental.pallas import tpu_sc as plsc


def _edge_mlp_body(he_ref, w1t_ref, b1_ref, w2t_ref, b2_ref, out_ref):
    x = he_ref[...]
    y = jnp.maximum(
        jnp.dot(x, w1t_ref[...], preferred_element_type=jnp.float32) + b1_ref[...],
        0.0,
    )
    z = (
        jnp.dot(y, w2t_ref[...], preferred_element_type=jnp.float32) + b2_ref[...]
    ).astype(jnp.bfloat16)
    zi = lax.bitcast_convert_type(z, jnp.int16)
    d = zi.shape[-1]
    lo = lax.convert_element_type(zi[:, : d // 2], jnp.int32) & 0xFFFF
    hi = lax.convert_element_type(zi[:, d // 2 :], jnp.int32) << 16
    out_ref[...] = lo | hi


def _edge_mlp(he, w1t, b1, w2t, b2, block_e=16000):
    e, d = he.shape
    grid = e // block_e
    return pl.pallas_call(
        _edge_mlp_body,
        grid=(grid,),
        in_specs=[
            pl.BlockSpec((block_e, d), lambda i: (i, 0)),
            pl.BlockSpec((d, d), lambda i: (0, 0)),
            pl.BlockSpec((1, d), lambda i: (0, 0)),
            pl.BlockSpec((d, d), lambda i: (0, 0)),
            pl.BlockSpec((1, d), lambda i: (0, 0)),
        ],
        out_specs=pl.BlockSpec((block_e, d // 2), lambda i: (i, 0)),
        out_shape=jax.ShapeDtypeStruct((e, d // 2), jnp.int32),
    )(he, w1t, b1, w2t, b2)


def _node_mlp_body(p0_ref, p1_ref, w3t_ref, b3_ref, w4t_ref, b4_ref, out_ref):
    agg = p0_ref[0] + p1_ref[0]
    y = jnp.maximum(
        jnp.dot(agg, w3t_ref[...], preferred_element_type=jnp.float32) + b3_ref[...],
        0.0,
    )
    out_ref[...] = (
        jnp.dot(y, w4t_ref[...], preferred_element_type=jnp.float32) + b4_ref[...]
    )


def _node_mlp(parts, n, w3t, b3, w4t, b4, block_n=5000):
    d = parts.shape[2]
    grid = n // block_n
    return pl.pallas_call(
        _node_mlp_body,
        grid=(grid,),
        in_specs=[
            pl.BlockSpec((1, block_n, d), lambda i: (0, i, 0)),
            pl.BlockSpec((1, block_n, d), lambda i: (1, i, 0)),
            pl.BlockSpec((d, d), lambda i: (0, 0)),
            pl.BlockSpec((1, d), lambda i: (0, 0)),
            pl.BlockSpec((d, d), lambda i: (0, 0)),
            pl.BlockSpec((1, d), lambda i: (0, 0)),
        ],
        out_specs=pl.BlockSpec((block_n, d), lambda i: (i, 0)),
        out_shape=jax.ShapeDtypeStruct((n, d), jnp.float32),
    )(parts, parts, w3t, b3, w4t, b4)


def _sc_gather_mul_scatter(h, src3, dst3, he2, zeros_nd):
    n, d = h.shape
    nw, nchunk, ch = src3.shape
    nc, ns = 2, 16
    nbuf = 5
    ngrp = nchunk // nbuf
    n_pad = 10112
    rows_per_tile = n_pad // ns

    mesh = plsc.VectorSubcoreMesh(core_axis_name="c", subcore_axis_name="s")

    @functools.partial(
        pl.kernel,
        mesh=mesh,
        out_type=jax.ShapeDtypeStruct((nc, n_pad, d), jnp.float32),
        scratch_types=[
            pltpu.VMEM((nbuf, ch), jnp.int32),
            pltpu.VMEM((nbuf, ch), jnp.int32),
            pltpu.VMEM((nbuf, ch, d), jnp.float32),
            pltpu.VMEM((nbuf, ch, d // 2), jnp.int32),
            pltpu.VMEM_SHARED((n_pad, d), jnp.float32),
            pltpu.SemaphoreType.DMA((nbuf,)),
            pltpu.SemaphoreType.DMA((nbuf,)),
            pltpu.SemaphoreType.DMA((nbuf,)),
            pltpu.SemaphoreType.DMA((nbuf,)),
            pltpu.SemaphoreType.DMA((nbuf,)),
        ],
    )
    def k(h_hbm, src_hbm, dst_hbm, he2_hbm, z_hbm, out_hbm,
          sidx, didx, hrows, herows, agg,
          sem_si, sem_di, sem_g, sem_e, sem_s):
        cid = lax.axis_index("c")
        sid = lax.axis_index("s")
        wid = sid * nc + cid
        ebase = wid * nchunk * ch
        row0 = sid * rows_per_tile

        def issue_idx(c, b):
            pltpu.async_copy(src_hbm.at[wid, c], sidx.at[b], sem_si.at[b])
            pltpu.async_copy(dst_hbm.at[wid, c], didx.at[b], sem_di.at[b])

        def wait_idx(c, b):
            pltpu.make_async_copy(
                src_hbm.at[wid, c], sidx.at[b], sem_si.at[b]
            ).wait()
            pltpu.make_async_copy(
                dst_hbm.at[wid, c], didx.at[b], sem_di.at[b]
            ).wait()

        def issue_in(c, b):
            pltpu.async_copy(h_hbm.at[sidx.at[b]], hrows.at[b], sem_g.at[b])
            pltpu.async_copy(
                he2_hbm.at[pl.ds(ebase + c * ch, ch)], herows.at[b], sem_e.at[b]
            )

        def wait_in(c, b):
            pltpu.make_async_copy(
                h_hbm.at[sidx.at[b]], hrows.at[b], sem_g.at[b]
            ).wait()
            pltpu.make_async_copy(
                he2_hbm.at[pl.ds(ebase + c * ch, ch)], herows.at[b], sem_e.at[b]
            ).wait()

        def issue_scat(c, b):
            pltpu.async_copy(
                hrows.at[b], agg.at[didx.at[b]], sem_s.at[b], add=True
            )

        def wait_scat(c, b):
            pltpu.make_async_copy(
                hrows.at[b], agg.at[didx.at[b]], sem_s.at[b]
            ).wait()

        def mul_buf(b):
            def mul_row(i, carry):
                for t in range(d // 32):
                    v = herows[b, i, pl.ds(t * 16, 16)]
                    lo = lax.bitcast_convert_type(lax.shift_left(v, 16), jnp.float32)
                    hi = lax.bitcast_convert_type(
                        lax.bitwise_and(v, jnp.int32(-65536)), jnp.float32
                    )
                    sl0 = pl.ds(t * 16, 16)
                    sl1 = pl.ds(d // 2 + t * 16, 16)
                    hrows[b, i, sl0] = hrows[b, i, sl0] * lo
                    hrows[b, i, sl1] = hrows[b, i, sl1] * hi
                return carry

            lax.fori_loop(0, ch, mul_row, 0, unroll=4)

        issue_idx(0, 0)
        issue_idx(1, 1)
        pltpu.sync_copy(
            z_hbm.at[pl.ds(0, rows_per_tile)],
            agg.at[pl.ds(row0, rows_per_tile)],
        )
        wait_idx(0, 0)
        issue_in(0, 0)
        plsc.subcore_barrier()

        def group(g, carry):
            for u in range(nbuf):
                c = g * nbuf + u

                @pl.when(c >= 3)
                def _():
                    wait_scat(c - 3, (u - 3) % nbuf)

                @pl.when(c + 2 < nchunk)
                def _():
                    issue_idx(c + 2, (u + 2) % nbuf)

                @pl.when(c + 1 < nchunk)
                def _():
                    wait_idx(c + 1, (u + 1) % nbuf)
                    issue_in(c + 1, (u + 1) % nbuf)

                wait_in(c, u)
                mul_buf(u)
                issue_scat(c, u)
            return carry

        lax.fori_loop(0, ngrp, group, 0)
        for t3 in range(3):
            c = nchunk - 3 + t3
            wait_scat(c, c % nbuf)
        plsc.subcore_barrier()

        pltpu.sync_copy(
            agg.at[pl.ds(row0, rows_per_tile)],
            out_hbm.at[cid, pl.ds(row0, rows_per_tile)],
        )

    return k(h, src3, dst3, he2, zeros_nd)


def kernel(h, edge_index, he, W1, b1, W2, b2, W3, b3, W4, b4):
    n, d = h.shape
    e = he.shape[0]
    nw, ch = 32, 16
    nchunk = e // (nw * ch)
    src3 = edge_index[0].reshape(nw, nchunk, ch)
    dst3 = edge_index[1].reshape(nw, nchunk, ch)
    he2 = _edge_mlp(he, W1.T, b1.reshape(1, d), W2.T, b2.reshape(1, d))
    zeros_nd = jnp.zeros((n, d), jnp.float32)
    parts = _sc_gather_mul_scatter(h, src3, dst3, he2, zeros_nd)
    out = _node_mlp(parts, n, W3.T, b3.reshape(1, d), W4.T, b4.reshape(1, d))
    return out
